# Initial kernel scaffold; baseline (speedup 1.0000x reference)
import jax
import jax.numpy as jnp
from jax.experimental import pallas as pl


def kernel(obj_sem_cls_pred, obj_labels, cur_step, total_steps):
    raise NotImplementedError("write your pallas kernel here")


# speedup vs baseline: 5.5925x; 5.5925x over previous
import jax
import jax.numpy as jnp
from jax.experimental import pallas as pl


def _onehot_body(lab_ref, out_ref):
    n = out_ref.shape[1]
    c = out_ref.shape[2]
    lab = lab_ref[0, 0, :]
    col = jax.lax.broadcasted_iota(jnp.int32, (n, c), 1)
    oh = (col == lab[:, None]) & (lab[:, None] >= 0)
    out_ref[0] = oh.astype(out_ref.dtype)


def kernel(obj_sem_cls_pred, obj_labels, cur_step, total_steps):
    B, N, C = obj_sem_cls_pred.shape
    labels = obj_labels.astype(jnp.int32).reshape(B, 1, N)
    return pl.pallas_call(
        _onehot_body,
        grid=(B,),
        in_specs=[pl.BlockSpec((1, 1, N), lambda b: (b, 0, 0))],
        out_specs=pl.BlockSpec((1, N, C), lambda b: (b, 0, 0)),
        out_shape=jax.ShapeDtypeStruct((B, N, C), obj_sem_cls_pred.dtype),
    )(labels)
